# Initial kernel scaffold; baseline (speedup 1.0000x reference)
import functools

import jax
import jax.numpy as jnp
from jax import lax
from jax.experimental import pallas as pl
from jax.experimental.pallas import tpu as pltpu
from jax.experimental.pallas import tpu_sc as plsc

NN = 100000
NP = 100352
NROW = NP // 128
NP16 = NP // 16
EE = 6400000
ROWS = EE // 128
NC, NS = 2, 16
NWK = NC * NS
RPW = ROWS // NWK
XROW0 = NWK * RPW
KW = 11
NWIN = RPW // KW

_MESH = plsc.VectorSubcoreMesh(core_axis_name="c", subcore_axis_name="s")
_F32 = jnp.float32


def _worker():
    cid = lax.axis_index("c")
    sid = lax.axis_index("s")
    return cid, sid, sid * NC + cid


def _zero_slice(stage, dst_shared, slc):
    def zb(i, carry):
        stage[pl.ds(i * 16, 16)] = jnp.zeros((16,), _F32)
        return carry
    lax.fori_loop(0, NP16 // 16, zb, 0)
    pltpu.sync_copy(stage, dst_shared.at[slc])


def _stage_in(src_hbm, dst_shared, stage, slc):
    pltpu.sync_copy(src_hbm.at[slc], stage)
    pltpu.sync_copy(stage, dst_shared.at[slc])


def _stage_out(src_shared, dst_hbm, stage, slc, cid):
    pltpu.sync_copy(src_shared.at[slc], stage)
    pltpu.sync_copy(stage, dst_hbm.at[cid].at[slc])


def _deg_body(dst_rows, out, acc, dst_idx, ones_v, stage, sem):
    cid, sid, wid = _worker()
    slc = pl.ds(sid * NP16, NP16)
    _zero_slice(stage, acc, slc)
    for i in range(8):
        ones_v[pl.ds(i * 16, 16)] = jnp.full((16,), 1.0, _F32)
    plsc.subcore_barrier()
    base = wid * RPW

    def win(w, carry):
        r0 = base + w * KW
        pltpu.sync_copy(dst_rows.at[pl.ds(r0, KW)], dst_idx)
        descs = [
            pltpu.async_copy(ones_v, acc.at[dst_idx.at[j]], sem, add=True)
            for j in range(KW)
        ]
        for d in descs:
            d.wait()
        return carry

    lax.fori_loop(0, NWIN, win, 0)

    @pl.when(wid < 16)
    def _extra():
        pltpu.sync_copy(dst_rows.at[XROW0 + wid], dst_idx.at[0])
        pltpu.sync_copy(ones_v, acc.at[dst_idx.at[0]], add=True)

    plsc.subcore_barrier()
    _stage_out(acc, out, stage, slc, cid)


_deg_pass = functools.partial(
    pl.kernel,
    _deg_body,
    out_type=jax.ShapeDtypeStruct((NC, NP), _F32),
    mesh=_MESH,
    scratch_types=[
        pltpu.VMEM_SHARED((NP,), _F32),
        pltpu.VMEM((KW, 128), jnp.int32),
        pltpu.VMEM((128,), _F32),
        pltpu.VMEM((NP16,), _F32),
        pltpu.SemaphoreType.DMA,
    ],
)()


def _seg1_body(src_rows, dst_rows, u_hbm, out, tbl, acc, src_idx, dst_idx,
               vals, stage, sem_g, sem_s):
    cid, sid, wid = _worker()
    slc = pl.ds(sid * NP16, NP16)
    _zero_slice(stage, acc, slc)
    _stage_in(u_hbm, tbl, stage, slc)
    plsc.subcore_barrier()
    base = wid * RPW

    def win(w, carry):
        r0 = base + w * KW
        pltpu.sync_copy(src_rows.at[pl.ds(r0, KW)], src_idx)
        pltpu.sync_copy(dst_rows.at[pl.ds(r0, KW)], dst_idx)
        gd = [
            pltpu.async_copy(tbl.at[src_idx.at[j]], vals.at[j], sem_g)
            for j in range(KW)
        ]
        for d in gd:
            d.wait()
        sd = [
            pltpu.async_copy(vals.at[j], acc.at[dst_idx.at[j]], sem_s, add=True)
            for j in range(KW)
        ]
        for d in sd:
            d.wait()
        return carry

    lax.fori_loop(0, NWIN, win, 0)

    @pl.when(wid < 16)
    def _extra():
        pltpu.sync_copy(src_rows.at[XROW0 + wid], src_idx.at[0])
        pltpu.sync_copy(dst_rows.at[XROW0 + wid], dst_idx.at[0])
        pltpu.sync_copy(tbl.at[src_idx.at[0]], vals.at[0])
        pltpu.sync_copy(vals.at[0], acc.at[dst_idx.at[0]], add=True)

    plsc.subcore_barrier()
    _stage_out(acc, out, stage, slc, cid)


_seg1_pass = functools.partial(
    pl.kernel,
    _seg1_body,
    out_type=jax.ShapeDtypeStruct((NC, NP), _F32),
    mesh=_MESH,
    scratch_types=[
        pltpu.VMEM_SHARED((NP,), _F32),
        pltpu.VMEM_SHARED((NP,), _F32),
        pltpu.VMEM((KW, 128), jnp.int32),
        pltpu.VMEM((KW, 128), jnp.int32),
        pltpu.VMEM((KW, 128), _F32),
        pltpu.VMEM((NP16,), _F32),
        pltpu.SemaphoreType.DMA,
        pltpu.SemaphoreType.DMA,
    ],
)()


def _seg2_body(src_rows, dst_rows, v0_hbm, v1_hbm, out0, out1, tbl0, tbl1,
               acc0, acc1, src_idx, dst_idx, vals0, vals1, stage,
               sem_g, sem_s):
    cid, sid, wid = _worker()
    slc = pl.ds(sid * NP16, NP16)
    _zero_slice(stage, acc0, slc)
    _zero_slice(stage, acc1, slc)
    _stage_in(v0_hbm, tbl0, stage, slc)
    _stage_in(v1_hbm, tbl1, stage, slc)
    plsc.subcore_barrier()
    base = wid * RPW

    def win(w, carry):
        r0 = base + w * KW
        pltpu.sync_copy(src_rows.at[pl.ds(r0, KW)], src_idx)
        pltpu.sync_copy(dst_rows.at[pl.ds(r0, KW)], dst_idx)
        gd = [
            pltpu.async_copy(tbl0.at[src_idx.at[j]], vals0.at[j], sem_g)
            for j in range(KW)
        ] + [
            pltpu.async_copy(tbl1.at[src_idx.at[j]], vals1.at[j], sem_g)
            for j in range(KW)
        ]
        for d in gd:
            d.wait()
        sd = [
            pltpu.async_copy(vals0.at[j], acc0.at[dst_idx.at[j]], sem_s, add=True)
            for j in range(KW)
        ] + [
            pltpu.async_copy(vals1.at[j], acc1.at[dst_idx.at[j]], sem_s, add=True)
            for j in range(KW)
        ]
        for d in sd:
            d.wait()
        return carry

    lax.fori_loop(0, NWIN, win, 0)

    @pl.when(wid < 16)
    def _extra():
        pltpu.sync_copy(src_rows.at[XROW0 + wid], src_idx.at[0])
        pltpu.sync_copy(dst_rows.at[XROW0 + wid], dst_idx.at[0])
        pltpu.sync_copy(tbl0.at[src_idx.at[0]], vals0.at[0])
        pltpu.sync_copy(tbl1.at[src_idx.at[0]], vals1.at[0])
        pltpu.sync_copy(vals0.at[0], acc0.at[dst_idx.at[0]], add=True)
        pltpu.sync_copy(vals1.at[0], acc1.at[dst_idx.at[0]], add=True)

    plsc.subcore_barrier()
    _stage_out(acc0, out0, stage, slc, cid)
    _stage_out(acc1, out1, stage, slc, cid)


_seg2_pass = functools.partial(
    pl.kernel,
    _seg2_body,
    out_type=(jax.ShapeDtypeStruct((NC, NP), _F32),
              jax.ShapeDtypeStruct((NC, NP), _F32)),
    mesh=_MESH,
    scratch_types=[
        pltpu.VMEM_SHARED((NP,), _F32),
        pltpu.VMEM_SHARED((NP,), _F32),
        pltpu.VMEM_SHARED((NP,), _F32),
        pltpu.VMEM_SHARED((NP,), _F32),
        pltpu.VMEM((KW, 128), jnp.int32),
        pltpu.VMEM((KW, 128), jnp.int32),
        pltpu.VMEM((KW, 128), _F32),
        pltpu.VMEM((KW, 128), _F32),
        pltpu.VMEM((NP16,), _F32),
        pltpu.SemaphoreType.DMA,
        pltpu.SemaphoreType.DMA,
    ],
)()


_VSPEC = pl.BlockSpec(memory_space=pltpu.MemorySpace.VMEM)
_SSPEC = pl.BlockSpec(memory_space=pltpu.MemorySpace.SMEM)


def _tc1_body(degp, xp, dinv, u):
    d = degp[0] + degp[1] + 1.0
    r = lax.rsqrt(d)
    dinv[...] = r
    u[...] = xp[...] * r


def _tc1(degp, xp):
    return pl.pallas_call(
        _tc1_body,
        out_shape=(jax.ShapeDtypeStruct((NROW, 128), _F32),
                   jax.ShapeDtypeStruct((NROW, 128), _F32)),
        in_specs=[_VSPEC, _VSPEC],
        out_specs=(_VSPEC, _VSPEC),
    )(degp, xp)


def _tc2_body(dinv_r, u_r, sp_r, w1_r, b1_r, w2_r, v0_r, v1_r):
    dinv = dinv_r[...]
    s = dinv * (sp_r[0] + sp_r[1] + u_r[...])
    g0 = jnp.zeros_like(s)
    g1 = jnp.zeros_like(s)
    for j in range(16):
        h = jnp.maximum(s * w1_r[0, j] + b1_r[0, j], 0.0)
        g0 = g0 + h * w2_r[j, 0]
        g1 = g1 + h * w2_r[j, 1]
    v0_r[...] = g0 * dinv
    v1_r[...] = g1 * dinv


def _tc2(dinv, u, sp, w1, b1, w2):
    return pl.pallas_call(
        _tc2_body,
        out_shape=(jax.ShapeDtypeStruct((NROW, 128), _F32),
                   jax.ShapeDtypeStruct((NROW, 128), _F32)),
        in_specs=[_VSPEC, _VSPEC, _VSPEC, _SSPEC, _SSPEC, _SSPEC],
        out_specs=(_VSPEC, _VSPEC),
    )(dinv, u, sp, w1, b1, w2)


def _tc3_body(dinv_r, v0_r, v1_r, t0_r, t1_r, b2_r, o0_r, o1_r):
    dinv = dinv_r[...]
    o0_r[...] = dinv * (t0_r[0] + t0_r[1] + v0_r[...]) + b2_r[0, 0]
    o1_r[...] = dinv * (t1_r[0] + t1_r[1] + v1_r[...]) + b2_r[0, 1]


def _tc3(dinv, v0, v1, t0, t1, b2):
    return pl.pallas_call(
        _tc3_body,
        out_shape=(jax.ShapeDtypeStruct((NROW, 128), _F32),
                   jax.ShapeDtypeStruct((NROW, 128), _F32)),
        in_specs=[_VSPEC, _VSPEC, _VSPEC, _VSPEC, _VSPEC, _SSPEC],
        out_specs=(_VSPEC, _VSPEC),
    )(dinv, v0, v1, t0, t1, b2)


def kernel(x, edge_index, W1, b1, W2, b2):
    src_rows = edge_index[0].reshape(ROWS, 128)
    dst_rows = edge_index[1].reshape(ROWS, 128)
    xp = jnp.pad(x[:, 0], (0, NP - NN)).reshape(NROW, 128)

    degp = _deg_pass(dst_rows).reshape(NC, NROW, 128)
    dinv, u = _tc1(degp, xp)

    sp = _seg1_pass(src_rows, dst_rows, u.reshape(NP)).reshape(NC, NROW, 128)
    v0, v1 = _tc2(dinv, u, sp, W1, b1.reshape(1, 16), W2)

    t0, t1 = _seg2_pass(src_rows, dst_rows, v0.reshape(NP), v1.reshape(NP))
    o0, o1 = _tc3(dinv, v0, v1, t0.reshape(NC, NROW, 128),
                  t1.reshape(NC, NROW, 128), b2.reshape(1, 2))

    return jnp.stack([o0.reshape(NP)[:NN], o1.reshape(NP)[:NN]], axis=1)


# speedup vs baseline: 135.8038x; 135.8038x over previous
import functools

import jax
import jax.numpy as jnp
from jax import lax
from jax.experimental import pallas as pl
from jax.experimental.pallas import tpu as pltpu
from jax.experimental.pallas import tpu_sc as plsc

NN = 100000
NP = 100352
NROW = NP // 128
NP16 = NP // 16
EE = 6400000
NC, NS = 2, 16
NWK = NC * NS
NG = EE // 1024
GPW = NG // NWK
EXTRA = NG - GPW * NWK

_MESH = plsc.VectorSubcoreMesh(core_axis_name="c", subcore_axis_name="s")
_F32 = jnp.float32


def _worker():
    cid = lax.axis_index("c")
    sid = lax.axis_index("s")
    return cid, sid, sid * NC + cid


def _group_range(wid):
    base = wid * GPW + jnp.minimum(wid, EXTRA)
    ng = GPW + jnp.where(wid < EXTRA, 1, 0)
    return base, ng


def _zero_slice(stage, dst_shared, slc):
    def zb(i, carry):
        stage[pl.ds(i * 16, 16)] = jnp.zeros((16,), _F32)
        return carry
    lax.fori_loop(0, NP16 // 16, zb, 0)
    pltpu.sync_copy(stage, dst_shared.at[slc])


def _stage_in(src_hbm, dst_shared, stage, slc):
    pltpu.sync_copy(src_hbm.at[slc], stage)
    pltpu.sync_copy(stage, dst_shared.at[slc])


def _stage_out(src_shared, dst_hbm, stage, slc, cid):
    pltpu.sync_copy(src_shared.at[slc], stage)
    pltpu.sync_copy(stage, dst_hbm.at[cid].at[slc])


def _deg_body(dst_rows, out, acc, dst_idx, ones_v, stage, sem):
    cid, sid, wid = _worker()
    slc = pl.ds(sid * NP16, NP16)
    _zero_slice(stage, acc, slc)
    for i in range(8):
        ones_v[pl.ds(i * 16, 16)] = jnp.full((16,), 1.0, _F32)
    plsc.subcore_barrier()
    base, ng = _group_range(wid)

    def grp(g, carry):
        pltpu.sync_copy(dst_rows.at[base + g], dst_idx)
        descs = [
            pltpu.async_copy(ones_v, acc.at[dst_idx.at[j]], sem, add=True)
            for j in range(8)
        ]
        for d in descs:
            d.wait()
        return carry

    lax.fori_loop(0, ng, grp, 0)
    plsc.subcore_barrier()
    _stage_out(acc, out, stage, slc, cid)


_deg_pass = functools.partial(
    pl.kernel,
    _deg_body,
    out_type=jax.ShapeDtypeStruct((NC, NP), _F32),
    mesh=_MESH,
    scratch_types=[
        pltpu.VMEM_SHARED((NP,), _F32),
        pltpu.VMEM((8, 128), jnp.int32),
        pltpu.VMEM((128,), _F32),
        pltpu.VMEM((NP16,), _F32),
        pltpu.SemaphoreType.DMA,
    ],
)()


def _seg1_body(src_rows, dst_rows, u_hbm, out, tbl, acc, src_idx, dst_idx,
               vals, stage, sem_g, sem_s):
    cid, sid, wid = _worker()
    slc = pl.ds(sid * NP16, NP16)
    _zero_slice(stage, acc, slc)
    _stage_in(u_hbm, tbl, stage, slc)
    plsc.subcore_barrier()
    base, ng = _group_range(wid)

    def grp(g, carry):
        pltpu.sync_copy(src_rows.at[base + g], src_idx)
        pltpu.sync_copy(dst_rows.at[base + g], dst_idx)
        gd = [
            pltpu.async_copy(tbl.at[src_idx.at[j]], vals.at[j], sem_g)
            for j in range(8)
        ]
        for d in gd:
            d.wait()
        sd = [
            pltpu.async_copy(vals.at[j], acc.at[dst_idx.at[j]], sem_s, add=True)
            for j in range(8)
        ]
        for d in sd:
            d.wait()
        return carry

    lax.fori_loop(0, ng, grp, 0)
    plsc.subcore_barrier()
    _stage_out(acc, out, stage, slc, cid)


_seg1_pass = functools.partial(
    pl.kernel,
    _seg1_body,
    out_type=jax.ShapeDtypeStruct((NC, NP), _F32),
    mesh=_MESH,
    scratch_types=[
        pltpu.VMEM_SHARED((NP,), _F32),
        pltpu.VMEM_SHARED((NP,), _F32),
        pltpu.VMEM((8, 128), jnp.int32),
        pltpu.VMEM((8, 128), jnp.int32),
        pltpu.VMEM((8, 128), _F32),
        pltpu.VMEM((NP16,), _F32),
        pltpu.SemaphoreType.DMA,
        pltpu.SemaphoreType.DMA,
    ],
)()


def _seg2_body(src_rows, dst_rows, v0_hbm, v1_hbm, out0, out1, tbl0, tbl1,
               acc0, acc1, src_idx, dst_idx, vals0, vals1, stage,
               sem_g, sem_s):
    cid, sid, wid = _worker()
    slc = pl.ds(sid * NP16, NP16)
    _zero_slice(stage, acc0, slc)
    _zero_slice(stage, acc1, slc)
    _stage_in(v0_hbm, tbl0, stage, slc)
    _stage_in(v1_hbm, tbl1, stage, slc)
    plsc.subcore_barrier()
    base, ng = _group_range(wid)

    def grp(g, carry):
        pltpu.sync_copy(src_rows.at[base + g], src_idx)
        pltpu.sync_copy(dst_rows.at[base + g], dst_idx)
        gd = [
            pltpu.async_copy(tbl0.at[src_idx.at[j]], vals0.at[j], sem_g)
            for j in range(8)
        ] + [
            pltpu.async_copy(tbl1.at[src_idx.at[j]], vals1.at[j], sem_g)
            for j in range(8)
        ]
        for d in gd:
            d.wait()
        sd = [
            pltpu.async_copy(vals0.at[j], acc0.at[dst_idx.at[j]], sem_s, add=True)
            for j in range(8)
        ] + [
            pltpu.async_copy(vals1.at[j], acc1.at[dst_idx.at[j]], sem_s, add=True)
            for j in range(8)
        ]
        for d in sd:
            d.wait()
        return carry

    lax.fori_loop(0, ng, grp, 0)
    plsc.subcore_barrier()
    _stage_out(acc0, out0, stage, slc, cid)
    _stage_out(acc1, out1, stage, slc, cid)


_seg2_pass = functools.partial(
    pl.kernel,
    _seg2_body,
    out_type=(jax.ShapeDtypeStruct((NC, NP), _F32),
              jax.ShapeDtypeStruct((NC, NP), _F32)),
    mesh=_MESH,
    scratch_types=[
        pltpu.VMEM_SHARED((NP,), _F32),
        pltpu.VMEM_SHARED((NP,), _F32),
        pltpu.VMEM_SHARED((NP,), _F32),
        pltpu.VMEM_SHARED((NP,), _F32),
        pltpu.VMEM((8, 128), jnp.int32),
        pltpu.VMEM((8, 128), jnp.int32),
        pltpu.VMEM((8, 128), _F32),
        pltpu.VMEM((8, 128), _F32),
        pltpu.VMEM((NP16,), _F32),
        pltpu.SemaphoreType.DMA,
        pltpu.SemaphoreType.DMA,
    ],
)()


_VSPEC = pl.BlockSpec(memory_space=pltpu.MemorySpace.VMEM)
_SSPEC = pl.BlockSpec(memory_space=pltpu.MemorySpace.SMEM)


def _tc1_body(degp, xp, dinv, u):
    d = degp[0] + degp[1] + 1.0
    r = lax.rsqrt(d)
    dinv[...] = r
    u[...] = xp[...] * r


def _tc1(degp, xp):
    return pl.pallas_call(
        _tc1_body,
        out_shape=(jax.ShapeDtypeStruct((NROW, 128), _F32),
                   jax.ShapeDtypeStruct((NROW, 128), _F32)),
        in_specs=[_VSPEC, _VSPEC],
        out_specs=(_VSPEC, _VSPEC),
    )(degp, xp)


def _tc2_body(dinv_r, u_r, sp_r, w1_r, b1_r, w2_r, v0_r, v1_r):
    dinv = dinv_r[...]
    s = dinv * (sp_r[0] + sp_r[1] + u_r[...])
    g0 = jnp.zeros_like(s)
    g1 = jnp.zeros_like(s)
    for j in range(16):
        h = jnp.maximum(s * w1_r[0, j] + b1_r[0, j], 0.0)
        g0 = g0 + h * w2_r[j, 0]
        g1 = g1 + h * w2_r[j, 1]
    v0_r[...] = g0 * dinv
    v1_r[...] = g1 * dinv


def _tc2(dinv, u, sp, w1, b1, w2):
    return pl.pallas_call(
        _tc2_body,
        out_shape=(jax.ShapeDtypeStruct((NROW, 128), _F32),
                   jax.ShapeDtypeStruct((NROW, 128), _F32)),
        in_specs=[_VSPEC, _VSPEC, _VSPEC, _SSPEC, _SSPEC, _SSPEC],
        out_specs=(_VSPEC, _VSPEC),
    )(dinv, u, sp, w1, b1, w2)


def _tc3_body(dinv_r, v0_r, v1_r, t0_r, t1_r, b2_r, o0_r, o1_r):
    dinv = dinv_r[...]
    o0_r[...] = dinv * (t0_r[0] + t0_r[1] + v0_r[...]) + b2_r[0, 0]
    o1_r[...] = dinv * (t1_r[0] + t1_r[1] + v1_r[...]) + b2_r[0, 1]


def _tc3(dinv, v0, v1, t0, t1, b2):
    return pl.pallas_call(
        _tc3_body,
        out_shape=(jax.ShapeDtypeStruct((NROW, 128), _F32),
                   jax.ShapeDtypeStruct((NROW, 128), _F32)),
        in_specs=[_VSPEC, _VSPEC, _VSPEC, _VSPEC, _VSPEC, _SSPEC],
        out_specs=(_VSPEC, _VSPEC),
    )(dinv, v0, v1, t0, t1, b2)


def kernel(x, edge_index, W1, b1, W2, b2):
    src_rows = edge_index[0].reshape(NG, 8, 128)
    dst_rows = edge_index[1].reshape(NG, 8, 128)
    xp = jnp.pad(x[:, 0], (0, NP - NN)).reshape(NROW, 128)

    degp = _deg_pass(dst_rows).reshape(NC, NROW, 128)
    dinv, u = _tc1(degp, xp)

    sp = _seg1_pass(src_rows, dst_rows, u.reshape(NP)).reshape(NC, NROW, 128)
    v0, v1 = _tc2(dinv, u, sp, W1, b1.reshape(1, 16), W2)

    t0, t1 = _seg2_pass(src_rows, dst_rows, v0.reshape(NP), v1.reshape(NP))
    o0, o1 = _tc3(dinv, v0, v1, t0.reshape(NC, NROW, 128),
                  t1.reshape(NC, NROW, 128), b2.reshape(1, 2))

    return jnp.stack([o0.reshape(NP)[:NN], o1.reshape(NP)[:NN]], axis=1)
